# baseline (speedup 1.0000x reference)
import functools

import jax
import jax.numpy as jnp
from jax import lax
from jax.experimental import pallas as pl
from jax.experimental.pallas import tpu as pltpu
from jax.experimental.pallas import tpu_sc as plsc

_N = 16384
_C = 1000
_BC = 2048
_NB = _N // _BC
_NITER = 3

_info = plsc.get_sparse_core_info()
_NC = _info.num_cores
_NS = _info.num_subcores
_NW = _NC * _NS
_BW = _N // _NW
_NCH = _BW // 128


def _main_block(x_ref, t_ref, loss_ref):
    x = x_ref[...]
    tgt = t_ref[0]
    m = jnp.max(x, axis=0, keepdims=True)

    def mich(_, t):
        sel = x > t
        k = jnp.sum(sel.astype(jnp.float32), axis=0, keepdims=True)
        s = jnp.sum(jnp.where(sel, x, 0.0), axis=0, keepdims=True)
        return (s - 1.0) / jnp.maximum(k, 1.0)

    t = lax.fori_loop(0, _NITER, mich, m - 1.0)

    rowid = lax.broadcasted_iota(jnp.int32, (_C, _BC), 0)
    gx = jnp.sum(jnp.where(rowid == tgt, x, 0.0), axis=0, keepdims=True)
    sel = x > t
    xs = jnp.where(sel, x, 0.0)
    k = jnp.sum(sel.astype(jnp.float32), axis=0, keepdims=True)
    s = jnp.sum(xs, axis=0, keepdims=True)
    q = jnp.sum(xs * xs, axis=0, keepdims=True)
    tau = (s - 1.0) / jnp.maximum(k, 1.0)
    sump = s - k * tau
    sump2 = q - (2.0 * tau) * s + k * (tau * tau)
    loss_ref[0] = 0.5 + 0.5 * sump2 + tau * sump - gx


_sc_mesh = plsc.VectorSubcoreMesh(core_axis_name="c", subcore_axis_name="s")


@functools.partial(
    pl.kernel,
    mesh=_sc_mesh,
    out_type=jax.ShapeDtypeStruct((_NW, 16), jnp.float32),
    scratch_types=[
        pltpu.VMEM((_NCH, 128), jnp.float32),
        pltpu.VMEM((16,), jnp.float32),
    ],
)
def _sc_reduce(loss_hbm, out_hbm, buf_v, acc_v):
    wid = lax.axis_index("s") * _NC + lax.axis_index("c")
    pltpu.sync_copy(loss_hbm.at[wid], buf_v)
    acc = jnp.zeros((16,), jnp.float32)
    for c in range(_NCH):
        for h in range(8):
            acc = acc + buf_v[c, pl.ds(h * 16, 16)]
    acc_v[...] = acc
    pltpu.sync_copy(acc_v, out_hbm.at[wid])


def _combine_block(p_ref, o_ref):
    o_ref[...] = jnp.sum(p_ref[...]).reshape(1, 1) * (1.0 / _N)


@jax.jit
def kernel(input, target):
    xt = input.T
    tgt3 = target.astype(jnp.int32).reshape(_NB, 1, _BC)

    loss = pl.pallas_call(
        _main_block,
        grid=(_NB,),
        in_specs=[
            pl.BlockSpec((_C, _BC), lambda b: (0, b)),
            pl.BlockSpec((1, 1, _BC), lambda b: (b, 0, 0)),
        ],
        out_specs=pl.BlockSpec((1, 1, _BC), lambda b: (b, 0, 0)),
        out_shape=jax.ShapeDtypeStruct((_NB, 1, _BC), jnp.float32),
        compiler_params=pltpu.CompilerParams(
            dimension_semantics=("parallel",),
        ),
    )(xt, tgt3)

    partials = _sc_reduce(loss.reshape(_NW, _NCH, 128))

    total = pl.pallas_call(
        _combine_block,
        in_specs=[pl.BlockSpec((_NW, 16), lambda: (0, 0))],
        out_specs=pl.BlockSpec((1, 1), lambda: (0, 0)),
        out_shape=jax.ShapeDtypeStruct((1, 1), jnp.float32),
    )(partials)
    return total[0, 0]


# speedup vs baseline: 1.0003x; 1.0003x over previous
import functools

import jax
import jax.numpy as jnp
from jax import lax
from jax.experimental import pallas as pl
from jax.experimental.pallas import tpu as pltpu
from jax.experimental.pallas import tpu_sc as plsc

_N = 16384
_C = 1000
_BC = 2048
_NB = _N // _BC
_NITER = 3

_NC = 2
_NS = 16
_NW = _NC * _NS
_BW = _N // _NW
_NCH = _BW // 128


def _main_block(x_ref, t_ref, loss_ref):
    x = x_ref[...]
    tgt = t_ref[0]
    m = jnp.max(x, axis=0, keepdims=True)

    def mich(_, t):
        sel = x > t
        k = jnp.sum(sel.astype(jnp.float32), axis=0, keepdims=True)
        s = jnp.sum(jnp.where(sel, x, 0.0), axis=0, keepdims=True)
        return (s - 1.0) / jnp.maximum(k, 1.0)

    t = lax.fori_loop(0, _NITER, mich, m - 1.0)

    rowid = lax.broadcasted_iota(jnp.int32, (_C, _BC), 0)
    gx = jnp.sum(jnp.where(rowid == tgt, x, 0.0), axis=0, keepdims=True)
    sel = x > t
    xs = jnp.where(sel, x, 0.0)
    k = jnp.sum(sel.astype(jnp.float32), axis=0, keepdims=True)
    s = jnp.sum(xs, axis=0, keepdims=True)
    q = jnp.sum(xs * xs, axis=0, keepdims=True)
    tau = (s - 1.0) / jnp.maximum(k, 1.0)
    sump = s - k * tau
    sump2 = q - (2.0 * tau) * s + k * (tau * tau)
    loss_ref[0] = 0.5 + 0.5 * sump2 + tau * sump - gx


def _sc_reduce_body(loss_hbm, out_hbm, buf_v, acc_v):
    wid = lax.axis_index("s") * _NC + lax.axis_index("c")
    pltpu.sync_copy(loss_hbm.at[wid], buf_v)
    acc = jnp.zeros((16,), jnp.float32)
    for c in range(_NCH):
        for h in range(8):
            acc = acc + buf_v[c, pl.ds(h * 16, 16)]
    acc_v[...] = acc
    pltpu.sync_copy(acc_v, out_hbm.at[wid])


@functools.cache
def _sc_reduce():
    mesh = plsc.VectorSubcoreMesh(core_axis_name="c", subcore_axis_name="s")
    return pl.kernel(
        _sc_reduce_body,
        mesh=mesh,
        out_type=jax.ShapeDtypeStruct((_NW, 16), jnp.float32),
        scratch_types=[
            pltpu.VMEM((_NCH, 128), jnp.float32),
            pltpu.VMEM((16,), jnp.float32),
        ],
    )


def _combine_block(p_ref, o_ref):
    o_ref[...] = jnp.sum(p_ref[...]).reshape(1, 1) * (1.0 / _N)


@jax.jit
def kernel(input, target):
    xt = input.T
    tgt3 = target.astype(jnp.int32).reshape(_NB, 1, _BC)

    loss = pl.pallas_call(
        _main_block,
        grid=(_NB,),
        in_specs=[
            pl.BlockSpec((_C, _BC), lambda b: (0, b)),
            pl.BlockSpec((1, 1, _BC), lambda b: (b, 0, 0)),
        ],
        out_specs=pl.BlockSpec((1, 1, _BC), lambda b: (b, 0, 0)),
        out_shape=jax.ShapeDtypeStruct((_NB, 1, _BC), jnp.float32),
        compiler_params=pltpu.CompilerParams(
            dimension_semantics=("parallel",),
        ),
    )(xt, tgt3)

    partials = _sc_reduce()(loss.reshape(_NW, _NCH, 128))

    total = pl.pallas_call(
        _combine_block,
        in_specs=[pl.BlockSpec((_NW, 16), lambda: (0, 0))],
        out_specs=pl.BlockSpec((1, 1), lambda: (0, 0)),
        out_shape=jax.ShapeDtypeStruct((1, 1), jnp.float32),
    )(partials)
    return total[0, 0]
